# baseline (speedup 1.0000x reference)
import functools

import jax
import jax.numpy as jnp
from jax import lax
from jax.experimental import pallas as pl
from jax.experimental.pallas import tpu as pltpu
from jax.experimental.pallas import tpu_sc as plsc

_INFO = plsc.get_sparse_core_info()
_NC = _INFO.num_cores
_NS = _INFO.num_subcores
_NW = _NC * _NS

_R = 8
_IDX_STAGE = 128


_PHASES = 4


def _embedding_gather(idx, weight, phase, b_out):
    b, h = idx.shape
    v, d = weight.shape
    rows_per_w = b_out // _NW
    n_chunks = rows_per_w // _R
    assert rows_per_w == _IDX_STAGE
    mesh = plsc.VectorSubcoreMesh(core_axis_name="c", subcore_axis_name="s")

    @functools.partial(
        pl.kernel,
        out_type=jax.ShapeDtypeStruct((b_out, h, d), jnp.float32),
        mesh=mesh,
        scratch_types=[
            pltpu.VMEM((_IDX_STAGE, h), jnp.int32),
            pltpu.VMEM((2, _R, h, d), jnp.float32),
            pltpu.SemaphoreType.DMA,
            pltpu.SemaphoreType.DMA,
        ],
    )
    def k(idx_hbm, table_hbm, out_hbm, idx_v, rows_v, gsem, osem):
        wid = lax.axis_index("s") * _NC + lax.axis_index("c")
        base = wid * rows_per_w
        in_base = phase * b_out + base
        chunks_per_stage = _IDX_STAGE // _R

        def stage_idx(stage):
            pltpu.sync_copy(
                idx_hbm.at[pl.ds(in_base + stage * _IDX_STAGE, _IDX_STAGE)],
                idx_v,
            )

        def fire_chunk(j, bf):
            for rr in range(_R):
                r = lax.rem(j, chunks_per_stage) * _R + rr
                va = idx_v[r, pl.ds(0, 16)]
                vb = idx_v[r, pl.ds(h - 16, 16)]
                for c in range(h):
                    i = va[c] if c < 16 else vb[c - (h - 16)]
                    pltpu.make_async_copy(
                        table_hbm.at[pl.ds(i, 1)],
                        rows_v.at[bf, rr, pl.ds(c, 1)],
                        gsem,
                    ).start()

        def chunk_wait(j, bf):
            pltpu.make_async_copy(
                out_hbm.at[pl.ds(base, _R)], rows_v.at[bf], gsem
            ).wait()

        def od(j, bf):
            dst = out_hbm.at[pl.ds(base + j * _R, _R)]
            return pltpu.make_async_copy(rows_v.at[bf], dst, osem)

        stage_idx(0)
        fire_chunk(0, 0)

        def body(j, carry):
            bf = lax.rem(j, 2)
            chunk_wait(j, bf)

            @pl.when(j >= 1)
            def _():
                od(j - 1, 1 - bf).wait()

            @pl.when(lax.rem(j + 1, chunks_per_stage) == 0)
            def _():
                @pl.when(j + 1 < n_chunks)
                def _():
                    stage_idx((j + 1) // chunks_per_stage)

            @pl.when(j + 1 < n_chunks)
            def _():
                fire_chunk(j + 1, 1 - bf)

            od(j, bf).start()
            return carry

        lax.fori_loop(0, n_chunks, body, 0)
        od(n_chunks - 1, (n_chunks - 1) % 2).wait()

    return k(idx, weight)


def kernel(input_, weight):
    idx = input_.astype(jnp.int32)
    b = idx.shape[0]
    b_out = b // _PHASES
    outs = [
        _embedding_gather(idx, weight, p, b_out) for p in range(_PHASES)
    ]
    return jnp.concatenate(outs, axis=0)


# speedup vs baseline: 1.0714x; 1.0714x over previous
import functools

import jax
import jax.numpy as jnp
from jax import lax
from jax.experimental import pallas as pl
from jax.experimental.pallas import tpu as pltpu
from jax.experimental.pallas import tpu_sc as plsc

_INFO = plsc.get_sparse_core_info()
_NC = _INFO.num_cores
_NS = _INFO.num_subcores
_NW = _NC * _NS

_R = 8
_IDX_STAGE = 128


def _embedding_gather(idx, weight):
    b, h = idx.shape
    v, d = weight.shape
    rows_per_w = b // _NW
    n_chunks = rows_per_w // _R
    mesh = plsc.VectorSubcoreMesh(core_axis_name="c", subcore_axis_name="s")

    @functools.partial(
        pl.kernel,
        out_type=jax.ShapeDtypeStruct((b, h, d), jnp.float32),
        mesh=mesh,
        scratch_types=[
            pltpu.VMEM((_IDX_STAGE, h), jnp.int32),
            pltpu.VMEM((2, _R, h, d), jnp.float32),
            pltpu.SemaphoreType.DMA((2,)),
            pltpu.SemaphoreType.DMA,
        ],
    )
    def k(idx_hbm, table_hbm, out_hbm, idx_v, rows_v, gsem, osem):
        wid = lax.axis_index("s") * _NC + lax.axis_index("c")
        base = wid * rows_per_w
        chunks_per_stage = _IDX_STAGE // _R

        def stage_idx(stage):
            pltpu.sync_copy(
                idx_hbm.at[pl.ds(base + stage * _IDX_STAGE, _IDX_STAGE)], idx_v
            )

        def fire_chunk(j, bf):
            for rr in range(_R):
                r = lax.rem(j, chunks_per_stage) * _R + rr
                va = idx_v[r, pl.ds(0, 16)]
                vb = idx_v[r, pl.ds(h - 16, 16)]
                for c in range(h):
                    i = va[c] if c < 16 else vb[c - (h - 16)]
                    pltpu.make_async_copy(
                        table_hbm.at[pl.ds(i, 1)],
                        rows_v.at[bf, rr, pl.ds(c, 1)],
                        gsem.at[bf],
                    ).start()

        def chunk_wait(j, bf):
            pltpu.make_async_copy(
                out_hbm.at[pl.ds(base, _R)], rows_v.at[bf], gsem.at[bf]
            ).wait()

        def od(j, bf):
            dst = out_hbm.at[pl.ds(base + j * _R, _R)]
            return pltpu.make_async_copy(rows_v.at[bf], dst, osem)

        stage_idx(0)
        fire_chunk(0, 0)

        def body(j, carry):
            bf = lax.rem(j, 2)

            @pl.when(j >= 1)
            def _():
                od(j - 1, 1 - bf).wait()

            @pl.when(lax.rem(j + 1, chunks_per_stage) == 0)
            def _():
                @pl.when(j + 1 < n_chunks)
                def _():
                    stage_idx((j + 1) // chunks_per_stage)

            @pl.when(j + 1 < n_chunks)
            def _():
                fire_chunk(j + 1, 1 - bf)

            chunk_wait(j, bf)
            od(j, bf).start()
            return carry

        lax.fori_loop(0, n_chunks, body, 0)
        od(n_chunks - 1, (n_chunks - 1) % 2).wait()

    return k(idx, weight)


def kernel(input_, weight):
    return _embedding_gather(input_.astype(jnp.int32), weight)


# speedup vs baseline: 1.0731x; 1.0016x over previous
import functools

import jax
import jax.numpy as jnp
from jax import lax
from jax.experimental import pallas as pl
from jax.experimental.pallas import tpu as pltpu
from jax.experimental.pallas import tpu_sc as plsc

_INFO = plsc.get_sparse_core_info()
_NC = _INFO.num_cores
_NS = _INFO.num_subcores
_NW = _NC * _NS

_R = 16
_IDX_STAGE = 128


def _embedding_gather(idx, weight):
    b, h = idx.shape
    v, d = weight.shape
    rows_per_w = b // _NW
    n_chunks = rows_per_w // _R
    mesh = plsc.VectorSubcoreMesh(core_axis_name="c", subcore_axis_name="s")

    @functools.partial(
        pl.kernel,
        out_type=jax.ShapeDtypeStruct((b, h, d), jnp.float32),
        mesh=mesh,
        scratch_types=[
            pltpu.VMEM((_IDX_STAGE, h), jnp.int32),
            pltpu.VMEM((2, _R, h, d), jnp.float32),
            pltpu.SemaphoreType.DMA((2,)),
            pltpu.SemaphoreType.DMA,
        ],
    )
    def k(idx_hbm, table_hbm, out_hbm, idx_v, rows_v, gsem, osem):
        wid = lax.axis_index("s") * _NC + lax.axis_index("c")
        base = wid * rows_per_w
        chunks_per_stage = _IDX_STAGE // _R

        def stage_idx(stage):
            pltpu.sync_copy(
                idx_hbm.at[pl.ds(base + stage * _IDX_STAGE, _IDX_STAGE)], idx_v
            )

        def fire_chunk(j, bf):
            for rr in range(_R):
                r = lax.rem(j, chunks_per_stage) * _R + rr
                va = idx_v[r, pl.ds(0, 16)]
                vb = idx_v[r, pl.ds(h - 16, 16)]
                for c in range(h):
                    i = va[c] if c < 16 else vb[c - (h - 16)]
                    pltpu.make_async_copy(
                        table_hbm.at[pl.ds(i, 1)],
                        rows_v.at[bf, rr, pl.ds(c, 1)],
                        gsem.at[bf],
                    ).start()

        def chunk_wait(j, bf):
            pltpu.make_async_copy(
                out_hbm.at[pl.ds(base, _R)], rows_v.at[bf], gsem.at[bf]
            ).wait()

        def od(j, bf):
            dst = out_hbm.at[pl.ds(base + j * _R, _R)]
            return pltpu.make_async_copy(rows_v.at[bf], dst, osem)

        stage_idx(0)
        fire_chunk(0, 0)

        def body(j, carry):
            bf = lax.rem(j, 2)

            @pl.when(j >= 1)
            def _():
                od(j - 1, 1 - bf).wait()

            @pl.when(lax.rem(j + 1, chunks_per_stage) == 0)
            def _():
                @pl.when(j + 1 < n_chunks)
                def _():
                    stage_idx((j + 1) // chunks_per_stage)

            @pl.when(j + 1 < n_chunks)
            def _():
                fire_chunk(j + 1, 1 - bf)

            chunk_wait(j, bf)
            od(j, bf).start()
            return carry

        lax.fori_loop(0, n_chunks, body, 0)
        od(n_chunks - 1, (n_chunks - 1) % 2).wait()

    return k(idx, weight)


def kernel(input_, weight):
    return _embedding_gather(input_.astype(jnp.int32), weight)
